# Initial kernel scaffold; baseline (speedup 1.0000x reference)
import functools

import jax
import jax.numpy as jnp
from jax import lax
from jax.experimental import pallas as pl
from jax.experimental.pallas import tpu as pltpu
from jax.experimental.pallas import tpu_sc as plsc

B = 16384
F = 26
V = 100000
D = 32
OUT = 64
EPS = 1e-5

NC = 2
NS = 16
NW = NC * NS
N = F * B
N_PER_W = N // NW
CH = 1024
NCH = N_PER_W // CH


def _sc_gather(idx3, tab2d):
    mesh = plsc.VectorSubcoreMesh(core_axis_name="c", subcore_axis_name="s")

    @functools.partial(
        pl.kernel,
        out_type=jax.ShapeDtypeStruct((N, D), jnp.float32),
        mesh=mesh,
        scratch_types=[
            pltpu.VMEM((NCH, CH), jnp.int32),
            pltpu.VMEM((CH, D), jnp.float32),
            pltpu.VMEM((CH, D), jnp.float32),
            pltpu.SemaphoreType.DMA,
            pltpu.SemaphoreType.DMA,
            pltpu.SemaphoreType.DMA,
            pltpu.SemaphoreType.DMA,
        ],
    )
    def k(idx_hbm, tab_hbm, out_hbm, idx_v, rows0, rows1, g0, g1, w0, w1):
        wid = lax.axis_index("s") * NC + lax.axis_index("c")
        base = wid * N_PER_W
        pltpu.sync_copy(idx_hbm.at[wid], idx_v)
        bufs = (rows0, rows1)
        gsems = (g0, g1)
        wsems = (w0, w1)
        gathers = [None, None]
        writes = [None, None]
        for j in range(NCH):
            bsel = j % 2
            if writes[bsel] is not None:
                writes[bsel].wait()
            gathers[bsel] = pltpu.async_copy(
                tab_hbm.at[idx_v.at[j]], bufs[bsel], gsems[bsel])
            if j >= 1:
                p = (j - 1) % 2
                gathers[p].wait()
                writes[p] = pltpu.async_copy(
                    bufs[p], out_hbm.at[pl.ds(base + (j - 1) * CH, CH)],
                    wsems[p])
        p = (NCH - 1) % 2
        gathers[p].wait()
        writes[p] = pltpu.async_copy(
            bufs[p], out_hbm.at[pl.ds(base + (NCH - 1) * CH, CH)], wsems[p])
        writes[p].wait()
        if NCH >= 2:
            writes[(NCH - 2) % 2].wait()

    return k(idx3, tab2d)


XB = 2048


def _tc_head_body(g_ref, w_ref, b_ref, ga_ref, be_ref, rm_ref, rv_ref, o_ref):
    acc = jnp.zeros((XB, OUT), jnp.float32)
    for i in range(F):
        acc += jnp.dot(g_ref[i], w_ref[i], preferred_element_type=jnp.float32)
    h = jnp.maximum(acc + b_ref[0], 0.0)
    scale = ga_ref[0] * lax.rsqrt(rv_ref[0] + EPS)
    shift = be_ref[0] - rm_ref[0] * scale
    o_ref[...] = h * scale + shift


def _tc_head(g3, wft, b, gamma, beta, rm, rv):
    return pl.pallas_call(
        _tc_head_body,
        grid=(B // XB,),
        in_specs=[
            pl.BlockSpec((F, XB, D), lambda i: (0, i, 0)),
            pl.BlockSpec((F, D, OUT), lambda i: (0, 0, 0)),
            pl.BlockSpec((1, OUT), lambda i: (0, 0)),
            pl.BlockSpec((1, OUT), lambda i: (0, 0)),
            pl.BlockSpec((1, OUT), lambda i: (0, 0)),
            pl.BlockSpec((1, OUT), lambda i: (0, 0)),
            pl.BlockSpec((1, OUT), lambda i: (0, 0)),
        ],
        out_specs=pl.BlockSpec((XB, OUT), lambda i: (i, 0)),
        out_shape=jax.ShapeDtypeStruct((B, OUT), jnp.float32),
    )(g3, wft, b, gamma, beta, rm, rv)


def kernel(cat0, cat1, cat2, cat3, cat4, cat5, cat6, cat7, cat8, cat9,
           cat10, cat11, cat12, cat13, cat14, cat15, cat16, cat17, cat18,
           cat19, cat20, cat21, cat22, cat23, cat24, cat25,
           tables, W, b, gamma, beta, running_mean, running_var):
    cats = jnp.stack(
        [cat0, cat1, cat2, cat3, cat4, cat5, cat6, cat7, cat8, cat9,
         cat10, cat11, cat12, cat13, cat14, cat15, cat16, cat17, cat18,
         cat19, cat20, cat21, cat22, cat23, cat24, cat25], axis=0)
    idx = cats + (jnp.arange(F, dtype=jnp.int32) * V)[:, None]
    idx3 = idx.reshape(NW, NCH, CH)
    tab2d = tables.reshape(F * V, D)
    g = _sc_gather(idx3, tab2d)
    g3 = g.reshape(F, B, D)
    wft = W.reshape(OUT, F, D).transpose(1, 2, 0)
    return _tc_head(g3, wft, b[None], gamma[None], beta[None],
                    running_mean[None], running_var[None])


# speedup vs baseline: 1.8075x; 1.8075x over previous
import functools

import jax
import jax.numpy as jnp
from jax import lax
from jax.experimental import pallas as pl
from jax.experimental.pallas import tpu as pltpu
from jax.experimental.pallas import tpu_sc as plsc

B = 16384
F = 26
V = 100000
D = 32
OUT = 64
EPS = 1e-5

NC = 2
NS = 16
NW = NC * NS
N = F * B
N_PER_W = N // NW
CH = 1024
NCH = N_PER_W // CH


def _sc_gather(idx3, tab2d):
    mesh = plsc.VectorSubcoreMesh(core_axis_name="c", subcore_axis_name="s")

    @functools.partial(
        pl.kernel,
        out_type=jax.ShapeDtypeStruct((N, D), jnp.float32),
        mesh=mesh,
        scratch_types=[
            pltpu.VMEM((N_PER_W,), jnp.int32),
            pltpu.VMEM((CH, D), jnp.float32),
            pltpu.VMEM((CH, D), jnp.float32),
            pltpu.SemaphoreType.DMA,
            pltpu.SemaphoreType.DMA,
            pltpu.SemaphoreType.DMA,
            pltpu.SemaphoreType.DMA,
        ],
        compiler_params=pltpu.CompilerParams(use_tc_tiling_on_sc=False),
    )
    def k(idx_hbm, tab_hbm, out_hbm, idx_v, rows0, rows1, g0, g1, w0, w1):
        wid = lax.axis_index("s") * NC + lax.axis_index("c")
        base = wid * N_PER_W
        pltpu.sync_copy(idx_hbm.at[pl.ds(base, N_PER_W)], idx_v)
        bufs = (rows0, rows1)
        gsems = (g0, g1)
        wsems = (w0, w1)
        gathers = [None, None]
        writes = [None, None]
        for j in range(NCH):
            bsel = j % 2
            if writes[bsel] is not None:
                writes[bsel].wait()
            gathers[bsel] = pltpu.async_copy(
                tab_hbm.at[idx_v.at[pl.ds(j * CH, CH)]], bufs[bsel],
                gsems[bsel])
            if j >= 1:
                p = (j - 1) % 2
                gathers[p].wait()
                writes[p] = pltpu.async_copy(
                    bufs[p], out_hbm.at[pl.ds(base + (j - 1) * CH, CH)],
                    wsems[p])
        p = (NCH - 1) % 2
        gathers[p].wait()
        writes[p] = pltpu.async_copy(
            bufs[p], out_hbm.at[pl.ds(base + (NCH - 1) * CH, CH)], wsems[p])
        writes[p].wait()
        if NCH >= 2:
            writes[(NCH - 2) % 2].wait()

    return k(idx3, tab2d)


XB = 2048


def _tc_head_body(g_ref, w_ref, b_ref, ga_ref, be_ref, rm_ref, rv_ref, o_ref):
    acc = jnp.zeros((XB, OUT), jnp.float32)
    for i in range(F):
        acc += jnp.dot(g_ref[i], w_ref[i], preferred_element_type=jnp.float32)
    h = jnp.maximum(acc + b_ref[0], 0.0)
    scale = ga_ref[0] * lax.rsqrt(rv_ref[0] + EPS)
    shift = be_ref[0] - rm_ref[0] * scale
    o_ref[...] = h * scale + shift


def _tc_head(g3, wft, b, gamma, beta, rm, rv):
    return pl.pallas_call(
        _tc_head_body,
        grid=(B // XB,),
        in_specs=[
            pl.BlockSpec((F, XB, D), lambda i: (0, i, 0)),
            pl.BlockSpec((F, D, OUT), lambda i: (0, 0, 0)),
            pl.BlockSpec((1, OUT), lambda i: (0, 0)),
            pl.BlockSpec((1, OUT), lambda i: (0, 0)),
            pl.BlockSpec((1, OUT), lambda i: (0, 0)),
            pl.BlockSpec((1, OUT), lambda i: (0, 0)),
            pl.BlockSpec((1, OUT), lambda i: (0, 0)),
        ],
        out_specs=pl.BlockSpec((XB, OUT), lambda i: (i, 0)),
        out_shape=jax.ShapeDtypeStruct((B, OUT), jnp.float32),
    )(g3, wft, b, gamma, beta, rm, rv)


def kernel(cat0, cat1, cat2, cat3, cat4, cat5, cat6, cat7, cat8, cat9,
           cat10, cat11, cat12, cat13, cat14, cat15, cat16, cat17, cat18,
           cat19, cat20, cat21, cat22, cat23, cat24, cat25,
           tables, W, b, gamma, beta, running_mean, running_var):
    cats = jnp.stack(
        [cat0, cat1, cat2, cat3, cat4, cat5, cat6, cat7, cat8, cat9,
         cat10, cat11, cat12, cat13, cat14, cat15, cat16, cat17, cat18,
         cat19, cat20, cat21, cat22, cat23, cat24, cat25], axis=0)
    idx = cats + (jnp.arange(F, dtype=jnp.int32) * V)[:, None]
    idx3 = idx.reshape(N)
    tab2d = tables.reshape(F * V, D)
    g = _sc_gather(idx3, tab2d)
    g3 = g.reshape(F, B, D)
    wft = W.reshape(OUT, F, D).transpose(1, 2, 0)
    return _tc_head(g3, wft, b[None], gamma[None], beta[None],
                    running_mean[None], running_var[None])


# speedup vs baseline: 1.8085x; 1.0006x over previous
import functools

import jax
import jax.numpy as jnp
from jax import lax
from jax.experimental import pallas as pl
from jax.experimental.pallas import tpu as pltpu
from jax.experimental.pallas import tpu_sc as plsc

B = 16384
F = 26
V = 100000
D = 32
OUT = 64
EPS = 1e-5

NC = 2
NS = 16
NW = NC * NS
N = F * B
N_PER_W = N // NW
CH = 1024
NCH = N_PER_W // CH


BPW = B // NW


def _sc_gather(idx_t, tables):
    mesh = plsc.VectorSubcoreMesh(core_axis_name="c", subcore_axis_name="s")

    @functools.partial(
        pl.kernel,
        out_type=jax.ShapeDtypeStruct((N, D), jnp.float32),
        mesh=mesh,
        scratch_types=[
            pltpu.VMEM((F * BPW,), jnp.int32),
            pltpu.VMEM((BPW, D), jnp.float32),
            pltpu.VMEM((BPW, D), jnp.float32),
            pltpu.SemaphoreType.DMA,
            pltpu.SemaphoreType.DMA,
            pltpu.SemaphoreType.DMA,
            pltpu.SemaphoreType.DMA,
        ],
        compiler_params=pltpu.CompilerParams(use_tc_tiling_on_sc=False),
    )
    def k(idx_hbm, tab_hbm, out_hbm, idx_v, rows0, rows1, g0, g1, w0, w1):
        wid = lax.axis_index("s") * NC + lax.axis_index("c")
        pltpu.sync_copy(idx_hbm.at[pl.ds(wid * F * BPW, F * BPW)], idx_v)
        bufs = (rows0, rows1)
        gsems = (g0, g1)
        wsems = (w0, w1)
        gathers = [None, None]
        writes = [None, None]
        for i in range(F):
            bsel = i % 2
            if writes[bsel] is not None:
                writes[bsel].wait()
            gathers[bsel] = pltpu.async_copy(
                tab_hbm.at[i].at[idx_v.at[pl.ds(i * BPW, BPW)]], bufs[bsel],
                gsems[bsel])
            if i >= 1:
                p = (i - 1) % 2
                gathers[p].wait()
                writes[p] = pltpu.async_copy(
                    bufs[p], out_hbm.at[pl.ds((i - 1) * B + wid * BPW, BPW)],
                    wsems[p])
        p = (F - 1) % 2
        gathers[p].wait()
        writes[p] = pltpu.async_copy(
            bufs[p], out_hbm.at[pl.ds((F - 1) * B + wid * BPW, BPW)], wsems[p])
        writes[p].wait()
        writes[(F - 2) % 2].wait()

    return k(idx_t, tables)


XB = 2048


def _tc_head_body(g_ref, w_ref, b_ref, ga_ref, be_ref, rm_ref, rv_ref, o_ref):
    acc = jnp.zeros((XB, OUT), jnp.float32)
    for i in range(F):
        acc += jnp.dot(g_ref[i], w_ref[i], preferred_element_type=jnp.float32)
    h = jnp.maximum(acc + b_ref[0], 0.0)
    scale = ga_ref[0] * lax.rsqrt(rv_ref[0] + EPS)
    shift = be_ref[0] - rm_ref[0] * scale
    o_ref[...] = h * scale + shift


def _tc_head(g3, wft, b, gamma, beta, rm, rv):
    return pl.pallas_call(
        _tc_head_body,
        grid=(B // XB,),
        in_specs=[
            pl.BlockSpec((F, XB, D), lambda i: (0, i, 0)),
            pl.BlockSpec((F, D, OUT), lambda i: (0, 0, 0)),
            pl.BlockSpec((1, OUT), lambda i: (0, 0)),
            pl.BlockSpec((1, OUT), lambda i: (0, 0)),
            pl.BlockSpec((1, OUT), lambda i: (0, 0)),
            pl.BlockSpec((1, OUT), lambda i: (0, 0)),
            pl.BlockSpec((1, OUT), lambda i: (0, 0)),
        ],
        out_specs=pl.BlockSpec((XB, OUT), lambda i: (i, 0)),
        out_shape=jax.ShapeDtypeStruct((B, OUT), jnp.float32),
    )(g3, wft, b, gamma, beta, rm, rv)


def kernel(cat0, cat1, cat2, cat3, cat4, cat5, cat6, cat7, cat8, cat9,
           cat10, cat11, cat12, cat13, cat14, cat15, cat16, cat17, cat18,
           cat19, cat20, cat21, cat22, cat23, cat24, cat25,
           tables, W, b, gamma, beta, running_mean, running_var):
    cats = jnp.stack(
        [cat0, cat1, cat2, cat3, cat4, cat5, cat6, cat7, cat8, cat9,
         cat10, cat11, cat12, cat13, cat14, cat15, cat16, cat17, cat18,
         cat19, cat20, cat21, cat22, cat23, cat24, cat25], axis=0)
    idx_t = cats.reshape(F, NW, BPW).transpose(1, 0, 2).reshape(N)
    g = _sc_gather(idx_t, tables)
    g3 = g.reshape(F, B, D)
    wft = W.reshape(OUT, F, D).transpose(1, 2, 0)
    return _tc_head(g3, wft, b[None], gamma[None], beta[None],
                    running_mean[None], running_var[None])


# speedup vs baseline: 2.8920x; 1.5991x over previous
import functools

import jax
import jax.numpy as jnp
from jax import lax
from jax.experimental import pallas as pl
from jax.experimental.pallas import tpu as pltpu
from jax.experimental.pallas import tpu_sc as plsc

B = 16384
F = 26
V = 100000
D = 32
OUT = 64
EPS = 1e-5

NC = 2
NS = 16
NW = NC * NS
HB = B // 2


def _sc_gather_t(idx_flat, tabT2):
    mesh = plsc.VectorSubcoreMesh(core_axis_name="c", subcore_axis_name="s")

    @functools.partial(
        pl.kernel,
        out_type=jax.ShapeDtypeStruct((F * D * B,), jnp.float32),
        mesh=mesh,
        scratch_types=[
            pltpu.VMEM((V,), jnp.float32),
            pltpu.VMEM((HB,), jnp.int32),
            pltpu.VMEM((HB,), jnp.float32),
        ],
        compiler_params=pltpu.CompilerParams(use_tc_tiling_on_sc=False,
                                             needs_layout_passes=False),
    )
    def k(idx_hbm, tab_hbm, out_hbm, row_v, idx_v, out_v):
        d = lax.axis_index("s") * NC + lax.axis_index("c")
        for i in range(F):
            r = i * D + d
            pltpu.sync_copy(tab_hbm.at[r], row_v)
            for h in range(2):
                pltpu.sync_copy(idx_hbm.at[pl.ds(i * B + h * HB, HB)], idx_v)

                def body(j, carry):
                    base = j * 64
                    for u in range(4):
                        o = base + u * 16
                        iv = idx_v[pl.ds(o, 16)]
                        out_v[pl.ds(o, 16)] = plsc.load_gather(row_v, [iv])
                    return carry

                lax.fori_loop(0, HB // 64, body, 0)
                pltpu.sync_copy(out_v, out_hbm.at[pl.ds(r * B + h * HB, HB)])

    return k(idx_flat, tabT2)


XB = 2048


def _tc_head_body(x_ref, w_ref, b_ref, ga_ref, be_ref, rm_ref, rv_ref, o_ref):
    acc = lax.dot_general(x_ref[...], w_ref[...],
                          dimension_numbers=(((0,), (1,)), ((), ())),
                          preferred_element_type=jnp.float32)
    h = jnp.maximum(acc + b_ref[0], 0.0)
    scale = ga_ref[0] * lax.rsqrt(rv_ref[0] + EPS)
    shift = be_ref[0] - rm_ref[0] * scale
    o_ref[...] = h * scale + shift


def _tc_head(xT, W, b, gamma, beta, rm, rv):
    return pl.pallas_call(
        _tc_head_body,
        grid=(B // XB,),
        in_specs=[
            pl.BlockSpec((F * D, XB), lambda i: (0, i)),
            pl.BlockSpec((OUT, F * D), lambda i: (0, 0)),
            pl.BlockSpec((1, OUT), lambda i: (0, 0)),
            pl.BlockSpec((1, OUT), lambda i: (0, 0)),
            pl.BlockSpec((1, OUT), lambda i: (0, 0)),
            pl.BlockSpec((1, OUT), lambda i: (0, 0)),
            pl.BlockSpec((1, OUT), lambda i: (0, 0)),
        ],
        out_specs=pl.BlockSpec((XB, OUT), lambda i: (i, 0)),
        out_shape=jax.ShapeDtypeStruct((B, OUT), jnp.float32),
    )(xT, W, b, gamma, beta, rm, rv)


def kernel(cat0, cat1, cat2, cat3, cat4, cat5, cat6, cat7, cat8, cat9,
           cat10, cat11, cat12, cat13, cat14, cat15, cat16, cat17, cat18,
           cat19, cat20, cat21, cat22, cat23, cat24, cat25,
           tables, W, b, gamma, beta, running_mean, running_var):
    cats = jnp.stack(
        [cat0, cat1, cat2, cat3, cat4, cat5, cat6, cat7, cat8, cat9,
         cat10, cat11, cat12, cat13, cat14, cat15, cat16, cat17, cat18,
         cat19, cat20, cat21, cat22, cat23, cat24, cat25], axis=0)
    idx_flat = cats.reshape(F * B)
    tabT2 = tables.transpose(0, 2, 1).reshape(F * D, V)
    g = _sc_gather_t(idx_flat, tabT2)
    xT = g.reshape(F * D, B)
    return _tc_head(xT, W, b[None], gamma[None], beta[None],
                    running_mean[None], running_var[None])


# speedup vs baseline: 2.9469x; 1.0190x over previous
import functools

import jax
import jax.numpy as jnp
from jax import lax
from jax.experimental import pallas as pl
from jax.experimental.pallas import tpu as pltpu
from jax.experimental.pallas import tpu_sc as plsc

B = 16384
F = 26
V = 100000
D = 32
OUT = 64
EPS = 1e-5

NC = 2
NS = 16
NW = NC * NS
HB = B // 2


def _sc_gather_t(idx_flat, tabT2):
    mesh = plsc.VectorSubcoreMesh(core_axis_name="c", subcore_axis_name="s")

    @functools.partial(
        pl.kernel,
        out_type=jax.ShapeDtypeStruct((F * D * B,), jnp.float32),
        mesh=mesh,
        scratch_types=[
            pltpu.VMEM((V,), jnp.float32),
            pltpu.VMEM((HB,), jnp.int32),
            pltpu.VMEM((HB,), jnp.float32),
            pltpu.VMEM((HB,), jnp.float32),
            pltpu.SemaphoreType.DMA,
            pltpu.SemaphoreType.DMA,
        ],
        compiler_params=pltpu.CompilerParams(use_tc_tiling_on_sc=False,
                                             needs_layout_passes=False),
    )
    def k(idx_hbm, tab_hbm, out_hbm, row_v, idx_v, out0, out1, os0, os1):
        d = lax.axis_index("s") * NC + lax.axis_index("c")
        outs = (out0, out1)
        osems = (os0, os1)
        wc = [None, None]
        for i in range(F):
            r = i * D + d
            pltpu.sync_copy(tab_hbm.at[r], row_v)
            for h in range(2):
                pltpu.sync_copy(idx_hbm.at[pl.ds(i * B + h * HB, HB)], idx_v)
                if wc[h] is not None:
                    wc[h].wait()
                out_v = outs[h]

                def body(j, carry):
                    base = j * 128
                    for u in range(8):
                        o = base + u * 16
                        iv = idx_v[pl.ds(o, 16)]
                        out_v[pl.ds(o, 16)] = plsc.load_gather(row_v, [iv])
                    return carry

                lax.fori_loop(0, HB // 128, body, 0)
                wc[h] = pltpu.async_copy(
                    out_v, out_hbm.at[pl.ds(r * B + h * HB, HB)], osems[h])
        wc[0].wait()
        wc[1].wait()

    return k(idx_flat, tabT2)


XB = 2048


def _tc_head_body(x_ref, w_ref, b_ref, ga_ref, be_ref, rm_ref, rv_ref, o_ref):
    acc = lax.dot_general(x_ref[...], w_ref[...],
                          dimension_numbers=(((0,), (1,)), ((), ())),
                          preferred_element_type=jnp.float32)
    h = jnp.maximum(acc + b_ref[0], 0.0)
    scale = ga_ref[0] * lax.rsqrt(rv_ref[0] + EPS)
    shift = be_ref[0] - rm_ref[0] * scale
    o_ref[...] = h * scale + shift


def _tc_head(xT, W, b, gamma, beta, rm, rv):
    return pl.pallas_call(
        _tc_head_body,
        grid=(B // XB,),
        in_specs=[
            pl.BlockSpec((F * D, XB), lambda i: (0, i)),
            pl.BlockSpec((OUT, F * D), lambda i: (0, 0)),
            pl.BlockSpec((1, OUT), lambda i: (0, 0)),
            pl.BlockSpec((1, OUT), lambda i: (0, 0)),
            pl.BlockSpec((1, OUT), lambda i: (0, 0)),
            pl.BlockSpec((1, OUT), lambda i: (0, 0)),
            pl.BlockSpec((1, OUT), lambda i: (0, 0)),
        ],
        out_specs=pl.BlockSpec((XB, OUT), lambda i: (i, 0)),
        out_shape=jax.ShapeDtypeStruct((B, OUT), jnp.float32),
    )(xT, W, b, gamma, beta, rm, rv)


def kernel(cat0, cat1, cat2, cat3, cat4, cat5, cat6, cat7, cat8, cat9,
           cat10, cat11, cat12, cat13, cat14, cat15, cat16, cat17, cat18,
           cat19, cat20, cat21, cat22, cat23, cat24, cat25,
           tables, W, b, gamma, beta, running_mean, running_var):
    cats = jnp.stack(
        [cat0, cat1, cat2, cat3, cat4, cat5, cat6, cat7, cat8, cat9,
         cat10, cat11, cat12, cat13, cat14, cat15, cat16, cat17, cat18,
         cat19, cat20, cat21, cat22, cat23, cat24, cat25], axis=0)
    idx_flat = cats.reshape(F * B)
    tabT2 = tables.transpose(0, 2, 1).reshape(F * D, V)
    g = _sc_gather_t(idx_flat, tabT2)
    xT = g.reshape(F * D, B)
    return _tc_head(xT, W, b[None], gamma[None], beta[None],
                    running_mean[None], running_var[None])


# speedup vs baseline: 6.2178x; 2.1100x over previous
import functools

import jax
import jax.numpy as jnp
from jax import lax
from jax.experimental import pallas as pl
from jax.experimental.pallas import tpu as pltpu
from jax.experimental.pallas import tpu_sc as plsc

B = 16384
F = 26
V = 100000
D = 32
OUT = 64
EPS = 1e-5

NC = 2
NS = 16
NW = NC * NS
HB = B // 2


def _sc_gather_t(idx_flat, tabT2):
    mesh = plsc.VectorSubcoreMesh(core_axis_name="c", subcore_axis_name="s")

    @functools.partial(
        pl.kernel,
        out_type=jax.ShapeDtypeStruct((F * D * B,), jnp.float32),
        mesh=mesh,
        scratch_types=[
            pltpu.VMEM((1, V), jnp.float32),
            pltpu.VMEM((HB,), jnp.int32),
            pltpu.VMEM((HB,), jnp.float32),
            pltpu.VMEM((HB,), jnp.float32),
            pltpu.SemaphoreType.DMA,
            pltpu.SemaphoreType.DMA,
        ],
        compiler_params=pltpu.CompilerParams(use_tc_tiling_on_sc=True,
                                             needs_layout_passes=False),
    )
    def k(idx_hbm, tab_hbm, out_hbm, row_v, idx_v, out0, out1, os0, os1):
        d = lax.axis_index("s") * NC + lax.axis_index("c")
        outs = (out0, out1)
        osems = (os0, os1)
        wc = [None, None]
        for i in range(F):
            r = i * D + d
            pltpu.sync_copy(tab_hbm.at[pl.ds(r, 1), :], row_v)
            for h in range(2):
                pltpu.sync_copy(idx_hbm.at[pl.ds(i * B + h * HB, HB)], idx_v)
                if wc[h] is not None:
                    wc[h].wait()
                out_v = outs[h]
                zz = jnp.zeros((16,), jnp.int32)

                def body(j, carry):
                    base = j * 128
                    for u in range(8):
                        o = base + u * 16
                        iv = idx_v[pl.ds(o, 16)]
                        out_v[pl.ds(o, 16)] = plsc.load_gather(row_v, [zz, iv])
                    return carry

                lax.fori_loop(0, HB // 128, body, 0)
                wc[h] = pltpu.async_copy(
                    out_v, out_hbm.at[pl.ds(r * B + h * HB, HB)], osems[h])
        wc[0].wait()
        wc[1].wait()

    return k(idx_flat, tabT2)


XB = 2048


def _tc_head_body(x_ref, w_ref, b_ref, ga_ref, be_ref, rm_ref, rv_ref, o_ref):
    acc = lax.dot_general(x_ref[...], w_ref[...],
                          dimension_numbers=(((0,), (1,)), ((), ())),
                          preferred_element_type=jnp.float32)
    h = jnp.maximum(acc + b_ref[0], 0.0)
    scale = ga_ref[0] * lax.rsqrt(rv_ref[0] + EPS)
    shift = be_ref[0] - rm_ref[0] * scale
    o_ref[...] = h * scale + shift


def _tc_head(xT, W, b, gamma, beta, rm, rv):
    return pl.pallas_call(
        _tc_head_body,
        grid=(B // XB,),
        in_specs=[
            pl.BlockSpec((F * D, XB), lambda i: (0, i)),
            pl.BlockSpec((OUT, F * D), lambda i: (0, 0)),
            pl.BlockSpec((1, OUT), lambda i: (0, 0)),
            pl.BlockSpec((1, OUT), lambda i: (0, 0)),
            pl.BlockSpec((1, OUT), lambda i: (0, 0)),
            pl.BlockSpec((1, OUT), lambda i: (0, 0)),
            pl.BlockSpec((1, OUT), lambda i: (0, 0)),
        ],
        out_specs=pl.BlockSpec((XB, OUT), lambda i: (i, 0)),
        out_shape=jax.ShapeDtypeStruct((B, OUT), jnp.float32),
    )(xT, W, b, gamma, beta, rm, rv)


def kernel(cat0, cat1, cat2, cat3, cat4, cat5, cat6, cat7, cat8, cat9,
           cat10, cat11, cat12, cat13, cat14, cat15, cat16, cat17, cat18,
           cat19, cat20, cat21, cat22, cat23, cat24, cat25,
           tables, W, b, gamma, beta, running_mean, running_var):
    cats = jnp.stack(
        [cat0, cat1, cat2, cat3, cat4, cat5, cat6, cat7, cat8, cat9,
         cat10, cat11, cat12, cat13, cat14, cat15, cat16, cat17, cat18,
         cat19, cat20, cat21, cat22, cat23, cat24, cat25], axis=0)
    idx_flat = cats.reshape(F * B)
    tabT2 = tables.transpose(0, 2, 1).reshape(F * D, V)
    g = _sc_gather_t(idx_flat, tabT2)
    xT = g.reshape(F * D, B)
    return _tc_head(xT, W, b[None], gamma[None], beta[None],
                    running_mean[None], running_var[None])
